# baseline (speedup 1.0000x reference)
import jax
import jax.numpy as jnp
from jax.experimental import pallas as pl
from jax.experimental.pallas import tpu as pltpu

_M = 4096
_N = 4096
_G = _M // 8
_CHUNKS = [16, 16, 32, 64, 64, 64, 64, 64, 64, 64]
assert sum(_CHUNKS) == _G
_MAXC = max(_CHUNKS)
_STARTS = [sum(_CHUNKS[:k]) for k in range(len(_CHUNKS))]


def _copy(A_ref, buf_ref, sem_ref, k):
    size = _CHUNKS[k]
    slot = k % 2
    return pltpu.make_async_copy(
        A_ref.at[pl.ds(_STARTS[k], size)],
        buf_ref.at[slot, pl.ds(0, size)],
        sem_ref.at[slot],
    )


def _loss_body(A_ref, xb_ref, b_ref, Iy_ref, x_ref, il_ref, iu_ref,
               l_ref, u_ref, out_ref, buf_ref, sem_ref):
    _copy(A_ref, buf_ref, sem_ref, 0).start()
    _copy(A_ref, buf_ref, sem_ref, 1).start()

    xv = x_ref[...]
    vv = (jnp.maximum(l_ref[...] - xv, 0.0) * il_ref[...]
          + jnp.maximum(xv - u_ref[...], 0.0) * iu_ref[...])
    bv = b_ref[...]
    total = jnp.sum(vv * vv)
    b_sq = jnp.sum(bv * bv)

    xb = xb_ref[...][None]
    for k, size in enumerate(_CHUNKS):
        _copy(A_ref, buf_ref, sem_ref, k).wait()
        blk = buf_ref[k % 2, pl.ds(0, size)]
        ax = jnp.sum(blk * xb, axis=2)
        bb = b_ref[pl.ds(_STARTS[k], size), :]
        cv = bb - ax
        cv = cv + jnp.maximum(-cv, 0.0) * Iy_ref[pl.ds(_STARTS[k], size), :]
        total = total + jnp.sum(cv * cv)
        if k + 2 < len(_CHUNKS):
            _copy(A_ref, buf_ref, sem_ref, k + 2).start()

    out_ref[0] = jnp.sqrt(total) / (1.0 + jnp.sqrt(b_sq))


def kernel(A, b, c, x, Iy, il, iu, l, u):
    del c
    A3 = A.reshape(_G, 8, _N)
    xb = jnp.broadcast_to(x.reshape(1, _N), (8, _N))
    b8 = b.reshape(_G, 8)
    Iy8 = Iy.reshape(_G, 8)
    small = [v.reshape(32, 128) for v in (x, il, iu, l, u)]
    full8 = pl.BlockSpec(memory_space=pltpu.VMEM)
    full = pl.BlockSpec(memory_space=pltpu.VMEM)
    out = pl.pallas_call(
        _loss_body,
        in_specs=[
            pl.BlockSpec(memory_space=pl.ANY),
            pl.BlockSpec(memory_space=pltpu.VMEM),
            full8,
            full8,
            full,
            full,
            full,
            full,
            full,
        ],
        out_specs=pl.BlockSpec(memory_space=pltpu.SMEM),
        out_shape=jax.ShapeDtypeStruct((1,), jnp.float32),
        scratch_shapes=[
            pltpu.VMEM((2, _MAXC, 8, _N), jnp.float32),
            pltpu.SemaphoreType.DMA((2,)),
        ],
    )(A3, xb, b8, Iy8, *small)
    return out[0]


# speedup vs baseline: 1.0089x; 1.0089x over previous
import jax
import jax.numpy as jnp
from jax.experimental import pallas as pl
from jax.experimental.pallas import tpu as pltpu

_M = 4096
_N = 4096
_G = _M // 8
_CHUNKS = [16, 16, 32, 64, 64, 64, 64, 64, 64, 64]
assert sum(_CHUNKS) == _G
_MAXC = max(_CHUNKS)
_STARTS = [sum(_CHUNKS[:k]) for k in range(len(_CHUNKS))]


def _copy(A_ref, buf_ref, sem_ref, k):
    size = _CHUNKS[k]
    slot = k % 3
    return pltpu.make_async_copy(
        A_ref.at[pl.ds(_STARTS[k], size)],
        buf_ref.at[slot, pl.ds(0, size)],
        sem_ref.at[slot],
    )


def _loss_body(A_ref, xb_ref, b_ref, Iy_ref, x_ref, il_ref, iu_ref,
               l_ref, u_ref, out_ref, buf_ref, sem_ref):
    _copy(A_ref, buf_ref, sem_ref, 0).start()
    _copy(A_ref, buf_ref, sem_ref, 1).start()

    xv = x_ref[...]
    vv = (jnp.maximum(l_ref[...] - xv, 0.0) * il_ref[...]
          + jnp.maximum(xv - u_ref[...], 0.0) * iu_ref[...])
    bv = b_ref[...]
    total = jnp.sum(vv * vv)
    b_sq = jnp.sum(bv * bv)

    xb = xb_ref[...][None]
    for k, size in enumerate(_CHUNKS):
        _copy(A_ref, buf_ref, sem_ref, k).wait()
        if k + 2 < len(_CHUNKS):
            _copy(A_ref, buf_ref, sem_ref, k + 2).start()
        blk = buf_ref[k % 3, pl.ds(0, size)]
        ax = jnp.sum(blk * xb, axis=2)
        bb = b_ref[pl.ds(_STARTS[k], size), :]
        cv = bb - ax
        cv = cv + jnp.maximum(-cv, 0.0) * Iy_ref[pl.ds(_STARTS[k], size), :]
        total = total + jnp.sum(cv * cv)

    out_ref[0] = jnp.sqrt(total) / (1.0 + jnp.sqrt(b_sq))


def kernel(A, b, c, x, Iy, il, iu, l, u):
    del c
    A3 = A.reshape(_G, 8, _N)
    xb = jnp.broadcast_to(x.reshape(1, _N), (8, _N))
    b8 = b.reshape(_G, 8)
    Iy8 = Iy.reshape(_G, 8)
    small = [v.reshape(32, 128) for v in (x, il, iu, l, u)]
    full8 = pl.BlockSpec(memory_space=pltpu.VMEM)
    full = pl.BlockSpec(memory_space=pltpu.VMEM)
    out = pl.pallas_call(
        _loss_body,
        in_specs=[
            pl.BlockSpec(memory_space=pl.ANY),
            pl.BlockSpec(memory_space=pltpu.VMEM),
            full8,
            full8,
            full,
            full,
            full,
            full,
            full,
        ],
        out_specs=pl.BlockSpec(memory_space=pltpu.SMEM),
        out_shape=jax.ShapeDtypeStruct((1,), jnp.float32),
        scratch_shapes=[
            pltpu.VMEM((3, _MAXC, 8, _N), jnp.float32),
            pltpu.SemaphoreType.DMA((3,)),
        ],
    )(A3, xb, b8, Iy8, *small)
    return out[0]
